# Initial kernel scaffold; baseline (speedup 1.0000x reference)
import jax
import jax.numpy as jnp
from jax.experimental import pallas as pl


def kernel(in_feas, codebook, Wr, br, cur_f, epoch):
    raise NotImplementedError("write your pallas kernel here")


# speedup vs baseline: 2.0784x; 2.0784x over previous
import math

import jax
import jax.numpy as jnp
from jax.experimental import pallas as pl
from jax.experimental.pallas import tpu as pltpu


def _sr_kernel(z_ref, cb_ref, wr_ref, br_ref, idx_ref, mask_ref, out_ref,
               cwr_ref):
    z = z_ref[0]
    cb = cb_ref[...]

    @pl.when(pl.program_id(0) == 0)
    def _():
        cwr_ref[...] = jax.lax.dot_general(
            cb, wr_ref[...], (((1,), (1,)), ((), ())),
            preferred_element_type=jnp.float32)

    zsq = jnp.sum(z * z, axis=1, keepdims=True)
    cbsq = jnp.sum(cb * cb, axis=1)[None, :]
    zc = jax.lax.dot_general(z, cb, (((1,), (1,)), ((), ())),
                             preferred_element_type=jnp.float32)
    d = zsq - 2.0 * zc + cbsq
    idx = jnp.argmin(d, axis=1).astype(jnp.int32)
    idx_ref[0, 0] = idx
    onehot = (jax.lax.broadcasted_iota(jnp.int32, d.shape, 1) == idx[:, None]
              ).astype(jnp.float32)
    logits = (jnp.dot(onehot, cwr_ref[...], preferred_element_type=jnp.float32)
              + br_ref[0][None, :])
    m = jax.nn.softmax(logits, axis=1)
    mask_ref[0] = m.T
    den = jnp.sum(m, axis=0) + 1e-6
    s = jax.lax.dot_general(onehot, m, (((0,), (0,)), ((), ())),
                            preferred_element_type=jnp.float32)
    numT = jax.lax.dot_general(s, cb, (((0,), (0,)), ((), ())),
                               preferred_element_type=jnp.float32)
    out_ref[0] = numT / den[:, None]


def kernel(in_feas, codebook, Wr, br, cur_f=1, epoch=0):
    Bb, Ll, Cc = in_feas.shape
    Kk = codebook.shape[0]
    Rr = Wr.shape[0]
    h = int(math.sqrt(Ll))
    w = Ll // h
    idx, mask, out = pl.pallas_call(
        _sr_kernel,
        grid=(Bb,),
        in_specs=[
            pl.BlockSpec((1, Ll, Cc), lambda b: (b, 0, 0)),
            pl.BlockSpec((Kk, Cc), lambda b: (0, 0)),
            pl.BlockSpec((Rr, Cc), lambda b: (0, 0)),
            pl.BlockSpec((1, Rr), lambda b: (0, 0)),
        ],
        out_specs=[
            pl.BlockSpec((1, 1, Ll), lambda b: (b, 0, 0)),
            pl.BlockSpec((1, Rr, Ll), lambda b: (b, 0, 0)),
            pl.BlockSpec((1, Rr, Cc), lambda b: (b, 0, 0)),
        ],
        out_shape=[
            jax.ShapeDtypeStruct((Bb, 1, Ll), jnp.int32),
            jax.ShapeDtypeStruct((Bb, Rr, Ll), jnp.float32),
            jax.ShapeDtypeStruct((Bb, Rr, Cc), jnp.float32),
        ],
        scratch_shapes=[pltpu.VMEM((Kk, Rr), jnp.float32)],
    )(in_feas, codebook, Wr, br.reshape(1, Rr))
    enc_idx = idx.reshape(Bb, h, w)
    region_mask = mask.reshape(Bb, Rr, h, w)
    return (out, enc_idx, region_mask)


# speedup vs baseline: 2.4674x; 1.1871x over previous
import math

import jax
import jax.numpy as jnp
from jax.experimental import pallas as pl
from jax.experimental.pallas import tpu as pltpu


def _sr_kernel(z_ref, cb_ref, cbt_ref, wr_ref, br_ref,
               idx_ref, mask_ref, out_ref,
               msmt_ref, cbsq_ref, s_ref):
    b = pl.program_id(0)
    nb = pl.num_programs(0)
    L = z_ref.shape[1]
    K = cb_ref.shape[0]
    R = wr_ref.shape[0]

    @pl.when(b == 0)
    def _init():
        cwr = jax.lax.dot_general(
            cb_ref[...], wr_ref[...], (((1,), (1,)), ((), ())),
            preferred_element_type=jnp.float32)
        logits = cwr + br_ref[0][None, :]
        mx = jnp.max(logits, axis=1, keepdims=True)
        e = jnp.exp(logits - mx)
        msm = e / jnp.sum(e, axis=1, keepdims=True)
        msmt_ref[...] = msm.T
        cbsq_ref[...] = jnp.sum(cbt_ref[...] * cbt_ref[...], axis=0,
                                keepdims=True)

    @pl.when(b < nb - 1)
    def _batch():
        z = z_ref[0]
        zsq = jnp.sum(z * z, axis=1, keepdims=True)
        zc = jax.lax.dot_general(
            z.astype(jnp.bfloat16), cb_ref[...].astype(jnp.bfloat16),
            (((1,), (1,)), ((), ())),
            preferred_element_type=jnp.float32)
        d = zsq - 2.0 * zc + cbsq_ref[...]
        dmin = jnp.min(d, axis=1, keepdims=True)
        iota = jax.lax.broadcasted_iota(jnp.int32, d.shape, 1)
        idx = jnp.min(jnp.where(d == dmin, iota, K),
                      axis=1).astype(jnp.int32)
        idx_ref[0, 0] = idx
        onehot_t = (jax.lax.broadcasted_iota(jnp.int32, (K, L), 0)
                    == idx[None, :]).astype(jnp.float32)
        msmt = msmt_ref[...]
        mask_t = jax.lax.dot_general(
            msmt, onehot_t, (((1,), (0,)), ((), ())),
            preferred_element_type=jnp.float32)
        mask_ref[0] = mask_t
        den = jnp.sum(mask_t, axis=1, keepdims=True) + 1e-6
        counts = jnp.sum(onehot_t, axis=1, keepdims=True).T
        s_rows = msmt * counts * (1.0 / den)
        s_ref[pl.ds(b * R, R), :] = s_rows

    @pl.when(b == nb - 1)
    def _final():
        num_t = jax.lax.dot_general(
            s_ref[...], cb_ref[...], (((1,), (0,)), ((), ())),
            preferred_element_type=jnp.float32)
        out_ref[...] = num_t.reshape(out_ref.shape)


def kernel(in_feas, codebook, Wr, br, cur_f=1, epoch=0):
    Bb, Ll, Cc = in_feas.shape
    Kk = codebook.shape[0]
    Rr = Wr.shape[0]
    h = int(math.sqrt(Ll))
    w = Ll // h
    cbt = codebook.T
    idx, mask, out = pl.pallas_call(
        _sr_kernel,
        grid=(Bb + 1,),
        in_specs=[
            pl.BlockSpec((1, Ll, Cc), lambda b: (jnp.minimum(b, Bb - 1), 0, 0)),
            pl.BlockSpec((Kk, Cc), lambda b: (0, 0)),
            pl.BlockSpec((Cc, Kk), lambda b: (0, 0)),
            pl.BlockSpec((Rr, Cc), lambda b: (0, 0)),
            pl.BlockSpec((1, Rr), lambda b: (0, 0)),
        ],
        out_specs=[
            pl.BlockSpec((1, 1, Ll), lambda b: (jnp.minimum(b, Bb - 1), 0, 0)),
            pl.BlockSpec((1, Rr, Ll), lambda b: (jnp.minimum(b, Bb - 1), 0, 0)),
            pl.BlockSpec((Bb, Rr, Cc), lambda b: (0, 0, 0)),
        ],
        out_shape=[
            jax.ShapeDtypeStruct((Bb, 1, Ll), jnp.int32),
            jax.ShapeDtypeStruct((Bb, Rr, Ll), jnp.float32),
            jax.ShapeDtypeStruct((Bb, Rr, Cc), jnp.float32),
        ],
        scratch_shapes=[
            pltpu.VMEM((Rr, Kk), jnp.float32),
            pltpu.VMEM((1, Kk), jnp.float32),
            pltpu.VMEM((Bb * Rr, Kk), jnp.float32),
        ],
    )(in_feas, codebook, cbt, Wr, br.reshape(1, Rr))
    enc_idx = idx.reshape(Bb, h, w)
    region_mask = mask.reshape(Bb, Rr, h, w)
    return (out, enc_idx, region_mask)
